# Initial kernel scaffold; baseline (speedup 1.0000x reference)
import jax
import jax.numpy as jnp
from jax.experimental import pallas as pl


def kernel(inputs, table):
    raise NotImplementedError("write your pallas kernel here")


# speedup vs baseline: 1.3031x; 1.3031x over previous
import functools

import jax
import jax.numpy as jnp
from jax import lax
from jax.experimental import pallas as pl
from jax.experimental.pallas import tpu as pltpu
from jax.experimental.pallas import tpu_sc as plsc

DIM = 32
GROUP = 128


@functools.lru_cache(maxsize=None)
def _make_gather(num_groups: int, k: int):
    info = plsc.get_sparse_core_info()
    nc, ns = info.num_cores, info.num_subcores
    nw = nc * ns
    groups_per_w = num_groups // nw
    assert groups_per_w * nw == num_groups
    iters = groups_per_w // k
    assert iters * k == groups_per_w and iters % 2 == 0

    mesh = plsc.VectorSubcoreMesh(core_axis_name="c", subcore_axis_name="s")

    @functools.partial(
        pl.kernel,
        mesh=mesh,
        compiler_params=pltpu.CompilerParams(use_tc_tiling_on_sc=False),
        out_type=jax.ShapeDtypeStruct((num_groups, GROUP, DIM), jnp.float32),
        scratch_types=[
            pltpu.VMEM((2, k, GROUP), jnp.int32),
            pltpu.VMEM((2, k, GROUP, DIM), jnp.float32),
            pltpu.SemaphoreType.DMA,
            pltpu.SemaphoreType.DMA,
            pltpu.SemaphoreType.DMA,
            pltpu.SemaphoreType.DMA,
        ],
    )
    def gather_kernel(idx_hbm, table_hbm, out_hbm, idx_v, rows_v,
                      gsem0, gsem1, wsem0, wsem1):
        gsems = (gsem0, gsem1)
        wsems = (wsem0, wsem1)
        wid = lax.axis_index("s") * nc + lax.axis_index("c")
        w0 = wid * groups_per_w

        def load_and_fire(i, b):
            g0 = w0 + i * k
            pltpu.sync_copy(idx_hbm.at[pl.ds(g0, k)], idx_v.at[b])
            for j in range(k):
                pltpu.async_copy(
                    table_hbm.at[idx_v.at[b].at[j]], rows_v.at[b].at[j], gsems[b])

        def wait_gather(b):
            pltpu.make_async_copy(
                out_hbm.at[pl.ds(0, k)], rows_v.at[b], gsems[b]).wait()

        def fire_writeback(i, b):
            g0 = w0 + i * k
            pltpu.async_copy(rows_v.at[b], out_hbm.at[pl.ds(g0, k)], wsems[b])

        def wait_writeback(b):
            pltpu.make_async_copy(
                rows_v.at[b], out_hbm.at[pl.ds(0, k)], wsems[b]).wait()

        load_and_fire(0, 0)

        def body(t, carry):
            for b in (0, 1):
                i = 2 * t + b
                nb = 1 - b

                @pl.when(i + 1 < iters)
                def _():
                    @pl.when(i >= 1)
                    def _():
                        wait_writeback(nb)

                    load_and_fire(i + 1, nb)

                wait_gather(b)
                fire_writeback(i, b)
            return carry

        lax.fori_loop(0, iters // 2, body, 0)
        wait_writeback(0)
        wait_writeback(1)

    return gather_kernel


def kernel(inputs, table):
    batch, hist = inputs.shape
    idx = jnp.asarray(inputs, jnp.int32).reshape(-1)
    num_groups = idx.shape[0] // GROUP
    idx2 = idx.reshape(num_groups, GROUP)
    out = _make_gather(num_groups, 10)(idx2, table)
    return out.reshape(batch, hist, DIM)


# speedup vs baseline: 1.7801x; 1.3661x over previous
import functools

import jax
import jax.numpy as jnp
from jax import lax
from jax.experimental import pallas as pl
from jax.experimental.pallas import tpu as pltpu
from jax.experimental.pallas import tpu_sc as plsc

DIM = 32


@functools.lru_cache(maxsize=None)
def _make_gather(batch: int, hist: int, cb: int):
    info = plsc.get_sparse_core_info()
    nc, ns = info.num_cores, info.num_subcores
    nw = nc * ns
    rows_per_w = batch // nw
    assert rows_per_w * nw == batch
    iters = rows_per_w // cb
    assert iters * cb == rows_per_w and iters % 2 == 0

    mesh = plsc.VectorSubcoreMesh(core_axis_name="c", subcore_axis_name="s")

    @functools.partial(
        pl.kernel,
        mesh=mesh,
        compiler_params=pltpu.CompilerParams(use_tc_tiling_on_sc=False),
        out_type=jax.ShapeDtypeStruct((batch, hist, DIM), jnp.float32),
        scratch_types=[
            pltpu.VMEM((2, cb, hist), jnp.int32),
            pltpu.VMEM((2, cb, hist, DIM), jnp.float32),
            pltpu.SemaphoreType.DMA,
            pltpu.SemaphoreType.DMA,
            pltpu.SemaphoreType.DMA,
            pltpu.SemaphoreType.DMA,
        ],
    )
    def gather_kernel(idx_hbm, table_hbm, out_hbm, idx_v, rows_v,
                      gsem0, gsem1, wsem0, wsem1):
        gsems = (gsem0, gsem1)
        wsems = (wsem0, wsem1)
        wid = lax.axis_index("s") * nc + lax.axis_index("c")
        w0 = wid * rows_per_w

        def load_and_fire(i, b):
            b0 = w0 + i * cb
            pltpu.sync_copy(idx_hbm.at[pl.ds(b0, cb)], idx_v.at[b])
            for j in range(cb):
                pltpu.async_copy(
                    table_hbm.at[idx_v.at[b].at[j]], rows_v.at[b].at[j], gsems[b])

        def wait_gather(b):
            pltpu.make_async_copy(
                out_hbm.at[pl.ds(0, cb)], rows_v.at[b], gsems[b]).wait()

        def fire_writeback(i, b):
            b0 = w0 + i * cb
            pltpu.async_copy(rows_v.at[b], out_hbm.at[pl.ds(b0, cb)], wsems[b])

        def wait_writeback(b):
            pltpu.make_async_copy(
                rows_v.at[b], out_hbm.at[pl.ds(0, cb)], wsems[b]).wait()

        load_and_fire(0, 0)

        def body(t, carry):
            for b in (0, 1):
                i = 2 * t + b
                nb = 1 - b

                @pl.when(i + 1 < iters)
                def _():
                    @pl.when(i >= 1)
                    def _():
                        wait_writeback(nb)

                    load_and_fire(i + 1, nb)

                wait_gather(b)
                fire_writeback(i, b)
            return carry

        lax.fori_loop(0, iters // 2, body, 0)
        wait_writeback(0)
        wait_writeback(1)

    return gather_kernel


def kernel(inputs, table):
    batch, hist = inputs.shape
    idx = jnp.asarray(inputs, jnp.int32)
    return _make_gather(batch, hist, 16)(idx, table)
